# baseline (speedup 1.0000x reference)
import functools

import jax
import jax.numpy as jnp
from jax import lax
from jax.experimental import pallas as pl
from jax.experimental.pallas import tpu as pltpu
from jax.experimental.pallas import tpu_sc as plsc

B = 16384
D = 64
UD = 128
NC = 2
NS = 16
NW = NC * NS
B_PER_W = B // NW
CHUNK = 128
NCHUNK = B_PER_W // CHUNK

BB = 2048


def _sc_gather(table, idx3):
    mesh = plsc.VectorSubcoreMesh(core_axis_name="c", subcore_axis_name="s")

    @functools.partial(
        pl.kernel,
        mesh=mesh,
        out_type=jax.ShapeDtypeStruct((B, D), jnp.float32),
        scratch_types=[
            pltpu.VMEM((NCHUNK, CHUNK), jnp.int32),
            pltpu.VMEM((B_PER_W, D), jnp.float32),
            pltpu.SemaphoreType.DMA,
        ],
        compiler_params=pltpu.CompilerParams(use_tc_tiling_on_sc=False),
    )
    def gather_kernel(table_hbm, idx_hbm, out_hbm, idx_v, rows_v, sem):
        wid = lax.axis_index("s") * NC + lax.axis_index("c")
        pltpu.sync_copy(idx_hbm.at[wid], idx_v)
        copies = [
            pltpu.async_copy(
                table_hbm.at[idx_v.at[j]],
                rows_v.at[pl.ds(j * CHUNK, CHUNK)],
                sem,
            )
            for j in range(NCHUNK)
        ]
        for c in copies:
            c.wait()
        pltpu.sync_copy(rows_v, out_hbm.at[pl.ds(wid * B_PER_W, B_PER_W)])

    return gather_kernel(table, idx3)


def _tc_mlp(user_emb, rows, Wu, bu2, W1u, W1i, b12, W2, b22, W3, b3s):
    grid = (B // BB,)

    def body(ue_ref, rows_ref, Wu_ref, bu_ref, W1u_ref, W1i_ref, b1_ref,
             W2_ref, b2_ref, W3_ref, b3_ref, out_ref):
        dn = (((1,), (1,)), ((), ()))
        u = lax.dot_general(ue_ref[...], Wu_ref[...], dn,
                            preferred_element_type=jnp.float32) + bu_ref[...]
        h1 = lax.dot_general(u, W1u_ref[...], dn,
                             preferred_element_type=jnp.float32)
        h1 = h1 + lax.dot_general(rows_ref[...], W1i_ref[...], dn,
                                  preferred_element_type=jnp.float32)
        h1 = jnp.maximum(h1 + b1_ref[...], 0.0)
        h2 = lax.dot_general(h1, W2_ref[...], dn,
                             preferred_element_type=jnp.float32)
        h2 = jnp.maximum(h2 + b2_ref[...], 0.0)
        o = lax.dot_general(h2, W3_ref[...], dn,
                            preferred_element_type=jnp.float32)
        out_ref[...] = o[:, 0] + b3_ref[0]

    full = lambda shape: pl.BlockSpec(shape, lambda i: (0, 0))
    return pl.pallas_call(
        body,
        grid=grid,
        in_specs=[
            pl.BlockSpec((BB, UD), lambda i: (i, 0)),
            pl.BlockSpec((BB, D), lambda i: (i, 0)),
            full((D, UD)),
            full((1, D)),
            full((D, D)),
            full((D, D)),
            full((1, D)),
            full((32, D)),
            full((1, 32)),
            full((1, 32)),
            pl.BlockSpec((1,), lambda i: (0,)),
        ],
        out_specs=pl.BlockSpec((BB,), lambda i: (i,)),
        out_shape=jax.ShapeDtypeStruct((B,), jnp.float32),
    )(user_emb, rows, Wu, bu2, W1u, W1i, b12, W2, b22, W3, b3s)


def kernel(user_emb, item_ids, Wu, bu, table, W1, b1, W2, b2, W3, b3):
    idx3 = item_ids.astype(jnp.int32).reshape(NW, NCHUNK, CHUNK)
    rows = _sc_gather(table, idx3)
    W1u = W1[:, :D]
    W1i = W1[:, D:]
    return _tc_mlp(user_emb, rows, Wu, bu.reshape(1, D), W1u, W1i,
                   b1.reshape(1, D), W2, b2.reshape(1, 32), W3, b3)


# speedup vs baseline: 2.1667x; 2.1667x over previous
import functools

import jax
import jax.numpy as jnp
from jax import lax
from jax.experimental import pallas as pl
from jax.experimental.pallas import tpu as pltpu
from jax.experimental.pallas import tpu_sc as plsc

B = 16384
D = 64
UD = 128
N_ITEMS = 1000000

NC = 2
NS = 16
NW = NC * NS
B_PER_W = B // NW
CHUNK = 128
NCHUNK = B_PER_W // CHUNK

BLKM = 4096
SPLIT = 122 * BLKM
GROWS = N_ITEMS - SPLIT
NBLK = -(-GROWS // BLKM)
BB = 2048


def _tc_relayout(tableT):
    def body(a_ref, b_ref, out_ref):
        a = a_ref[...]
        b = b_ref[...]
        out_ref[...] = jnp.concatenate(
            [jnp.transpose(a), jnp.transpose(b)], axis=1)

    return pl.pallas_call(
        body,
        grid=(NBLK,),
        in_specs=[
            pl.BlockSpec((D, BLKM), lambda i: (0, i)),
            pl.BlockSpec((D, BLKM), lambda i: (0, i + 122)),
        ],
        out_specs=pl.BlockSpec((BLKM, UD), lambda i: (i, 0)),
        out_shape=jax.ShapeDtypeStruct((GROWS, UD), jnp.float32),
    )(tableT, tableT)


def _sc_gather(g, idx3):
    mesh = plsc.VectorSubcoreMesh(core_axis_name="c", subcore_axis_name="s")

    @functools.partial(
        pl.kernel,
        mesh=mesh,
        out_type=jax.ShapeDtypeStruct((B, UD), jnp.float32),
        scratch_types=[
            pltpu.VMEM((NCHUNK, CHUNK), jnp.int32),
            pltpu.VMEM((B_PER_W, UD), jnp.float32),
            pltpu.SemaphoreType.DMA,
        ],
    )
    def gather_kernel(g_hbm, idx_hbm, out_hbm, idx_v, rows_v, sem):
        wid = lax.axis_index("s") * NC + lax.axis_index("c")
        pltpu.sync_copy(idx_hbm.at[wid], idx_v)
        copies = [
            pltpu.async_copy(
                g_hbm.at[idx_v.at[j]],
                rows_v.at[pl.ds(j * CHUNK, CHUNK)],
                sem,
            )
            for j in range(NCHUNK)
        ]
        for c in copies:
            c.wait()
        pltpu.sync_copy(rows_v, out_hbm.at[pl.ds(wid * B_PER_W, B_PER_W)])

    return gather_kernel(g, idx3)


def _tc_mlp(user_emb, rows, hi_flag, Wu, bu2, W1u, W1i, b12, W2, b22, W3, b3s):
    grid = (B // BB,)

    def body(ue_ref, rows_ref, flag_ref, Wu_ref, bu_ref, W1u_ref, W1i_ref,
             b1_ref, W2_ref, b2_ref, W3_ref, b3_ref, out_ref):
        dn = (((1,), (1,)), ((), ()))
        r = rows_ref[...]
        hi = flag_ref[...]
        i_emb = jnp.where(hi == 1, r[:, D:], r[:, :D])
        u = lax.dot_general(ue_ref[...], Wu_ref[...], dn,
                            preferred_element_type=jnp.float32) + bu_ref[...]
        h1 = lax.dot_general(u, W1u_ref[...], dn,
                             preferred_element_type=jnp.float32)
        h1 = h1 + lax.dot_general(i_emb, W1i_ref[...], dn,
                                  preferred_element_type=jnp.float32)
        h1 = jnp.maximum(h1 + b1_ref[...], 0.0)
        h2 = lax.dot_general(h1, W2_ref[...], dn,
                             preferred_element_type=jnp.float32)
        h2 = jnp.maximum(h2 + b2_ref[...], 0.0)
        o = lax.dot_general(h2, W3_ref[...], dn,
                            preferred_element_type=jnp.float32)
        out_ref[...] = o[:, 0] + b3_ref[0]

    full = lambda shape: pl.BlockSpec(shape, lambda i: (0, 0))
    return pl.pallas_call(
        body,
        grid=grid,
        in_specs=[
            pl.BlockSpec((BB, UD), lambda i: (i, 0)),
            pl.BlockSpec((BB, UD), lambda i: (i, 0)),
            pl.BlockSpec((BB, 1), lambda i: (i, 0)),
            full((D, UD)),
            full((1, D)),
            full((D, D)),
            full((D, D)),
            full((1, D)),
            full((32, D)),
            full((1, 32)),
            full((1, 32)),
            pl.BlockSpec((1,), lambda i: (0,)),
        ],
        out_specs=pl.BlockSpec((BB,), lambda i: (i,)),
        out_shape=jax.ShapeDtypeStruct((B,), jnp.float32),
    )(user_emb, rows, hi_flag, Wu, bu2, W1u, W1i, b12, W2, b22, W3, b3s)


def kernel(user_emb, item_ids, Wu, bu, table, W1, b1, W2, b2, W3, b3):
    ids = item_ids.astype(jnp.int32)
    hi_flag = (ids >= SPLIT).astype(jnp.int32).reshape(B, 1)
    idx = jnp.where(ids >= SPLIT, ids - SPLIT, ids)
    idx3 = idx.reshape(NW, NCHUNK, CHUNK)

    g = _tc_relayout(table.T)
    rows = _sc_gather(g, idx3)

    W1u = W1[:, :D]
    W1i = W1[:, D:]
    return _tc_mlp(user_emb, rows, hi_flag, Wu, bu.reshape(1, D), W1u, W1i,
                   b1.reshape(1, D), W2, b2.reshape(1, 32), W3, b3)


# speedup vs baseline: 3.1310x; 1.4451x over previous
import functools

import jax
import jax.numpy as jnp
from jax import lax
from jax.experimental import pallas as pl
from jax.experimental.pallas import tpu as pltpu
from jax.experimental.pallas import tpu_sc as plsc

B = 16384
D = 64
UD = 128
N_ITEMS = 1000000

NC = 2
NS = 16
NW = NC * NS
B_PER_W = B // NW
CHUNK = 128
NCHUNK = B_PER_W // CHUNK

BLKM = 4096
NBLK = 62
QROWS = NBLK * BLKM
BB = 2048


def _tc_relayout(tableT, eye):
    def body(x0_ref, x1_ref, x2_ref, x3_ref, eye_ref, out_ref):
        del eye_ref
        ys = [
            jnp.transpose(x_ref[...])
            for x_ref in (x0_ref, x1_ref, x2_ref, x3_ref)
        ]
        b0, b1, b2, b3 = [
            lax.bitcast_convert_type(y.astype(jnp.bfloat16), jnp.uint16)
            .astype(jnp.uint32)
            for y in ys
        ]
        p01 = lax.bitcast_convert_type((b1 << 16) | b0, jnp.float32)
        p23 = lax.bitcast_convert_type((b3 << 16) | b2, jnp.float32)
        out_ref[...] = jnp.concatenate([p01, p23], axis=1)

    return pl.pallas_call(
        body,
        grid=(NBLK,),
        in_specs=[
            pl.BlockSpec((D, BLKM), lambda i: (0, i)),
            pl.BlockSpec((D, BLKM), lambda i: (0, i + NBLK)),
            pl.BlockSpec((D, BLKM), lambda i: (0, i + 2 * NBLK)),
            pl.BlockSpec((D, BLKM),
                         lambda i: (0, jnp.minimum(i + 3 * NBLK, 244))),
            pl.BlockSpec((D, D), lambda i: (0, 0)),
        ],
        out_specs=pl.BlockSpec((BLKM, UD), lambda i: (i, 0)),
        out_shape=jax.ShapeDtypeStruct((QROWS, UD), jnp.float32),
    )(tableT, tableT, tableT, tableT, eye)


def _sc_gather(g, idx3):
    mesh = plsc.VectorSubcoreMesh(core_axis_name="c", subcore_axis_name="s")

    @functools.partial(
        pl.kernel,
        mesh=mesh,
        out_type=jax.ShapeDtypeStruct((B, UD), jnp.float32),
        scratch_types=[
            pltpu.VMEM((NCHUNK, CHUNK), jnp.int32),
            pltpu.VMEM((B_PER_W, UD), jnp.float32),
            pltpu.SemaphoreType.DMA,
        ],
    )
    def gather_kernel(g_hbm, idx_hbm, out_hbm, idx_v, rows_v, sem):
        wid = lax.axis_index("s") * NC + lax.axis_index("c")
        pltpu.sync_copy(idx_hbm.at[wid], idx_v)
        copies = [
            pltpu.async_copy(
                g_hbm.at[idx_v.at[j]],
                rows_v.at[pl.ds(j * CHUNK, CHUNK)],
                sem,
            )
            for j in range(NCHUNK)
        ]
        for c in copies:
            c.wait()
        pltpu.sync_copy(rows_v, out_hbm.at[pl.ds(wid * B_PER_W, B_PER_W)])

    return gather_kernel(g, idx3)


def _tc_mlp(user_emb, rows, quarter, Wu, bu2, W1u, W1i, b12, W2, b22, W3, b3s):
    grid = (B // BB,)

    def body(ue_ref, rows_ref, q_ref, Wu_ref, bu_ref, W1u_ref, W1i_ref,
             b1_ref, W2_ref, b2_ref, W3_ref, b3_ref, out_ref):
        dn = (((1,), (1,)), ((), ()))
        r = lax.bitcast_convert_type(rows_ref[...], jnp.uint32)
        q = q_ref[...]
        sel = jnp.where(q >= 2, r[:, D:], r[:, :D])
        lo = lax.bitcast_convert_type(
            (sel & 0xFFFF).astype(jnp.uint16), jnp.bfloat16)
        hi = lax.bitcast_convert_type(
            (sel >> 16).astype(jnp.uint16), jnp.bfloat16)
        i_emb = jnp.where((q & 1) == 1, hi, lo).astype(jnp.float32)
        u = lax.dot_general(ue_ref[...], Wu_ref[...], dn,
                            preferred_element_type=jnp.float32) + bu_ref[...]
        h1 = lax.dot_general(u, W1u_ref[...], dn,
                             preferred_element_type=jnp.float32)
        h1 = h1 + lax.dot_general(i_emb, W1i_ref[...], dn,
                                  preferred_element_type=jnp.float32)
        h1 = jnp.maximum(h1 + b1_ref[...], 0.0)
        h2 = lax.dot_general(h1, W2_ref[...], dn,
                             preferred_element_type=jnp.float32)
        h2 = jnp.maximum(h2 + b2_ref[...], 0.0)
        o = lax.dot_general(h2, W3_ref[...], dn,
                            preferred_element_type=jnp.float32)
        out_ref[...] = o[:, 0] + b3_ref[0]

    full = lambda shape: pl.BlockSpec(shape, lambda i: (0, 0))
    return pl.pallas_call(
        body,
        grid=grid,
        in_specs=[
            pl.BlockSpec((BB, UD), lambda i: (i, 0)),
            pl.BlockSpec((BB, UD), lambda i: (i, 0)),
            pl.BlockSpec((BB, 1), lambda i: (i, 0)),
            full((D, UD)),
            full((1, D)),
            full((D, D)),
            full((D, D)),
            full((1, D)),
            full((32, D)),
            full((1, 32)),
            full((1, 32)),
            pl.BlockSpec((1,), lambda i: (0,)),
        ],
        out_specs=pl.BlockSpec((BB,), lambda i: (i,)),
        out_shape=jax.ShapeDtypeStruct((B,), jnp.float32),
    )(user_emb, rows, quarter, Wu, bu2, W1u, W1i, b12, W2, b22, W3, b3s)


def kernel(user_emb, item_ids, Wu, bu, table, W1, b1, W2, b2, W3, b3):
    ids = item_ids.astype(jnp.int32)
    quarter = (ids // QROWS).astype(jnp.int32)
    idx = ids - quarter * QROWS
    idx3 = idx.reshape(NW, NCHUNK, CHUNK)

    g = _tc_relayout(table.T, jnp.eye(D, dtype=jnp.float32))
    rows = _sc_gather(g, idx3)

    W1u = W1[:, :D]
    W1i = W1[:, D:]
    return _tc_mlp(user_emb, rows, quarter.reshape(B, 1), Wu,
                   bu.reshape(1, D), W1u, W1i, b1.reshape(1, D), W2,
                   b2.reshape(1, 32), W3, b3)


# speedup vs baseline: 3.5340x; 1.1287x over previous
import functools

import jax
import jax.numpy as jnp
from jax import lax
from jax.experimental import pallas as pl
from jax.experimental.pallas import tpu as pltpu
from jax.experimental.pallas import tpu_sc as plsc

B = 16384
D = 64
UD = 128
N_ITEMS = 1000000

NC = 2
NS = 16
NW = NC * NS
B_PER_W = B // NW
CHUNK = 128
NCHUNK = B_PER_W // CHUNK

BLKM = 8192
NBLK = 31
QROWS = NBLK * BLKM
BB = 2048


def _tc_relayout(tableT, eye):
    def body(x0_ref, x1_ref, x2_ref, x3_ref, eye_ref, out_ref):
        del eye_ref
        u0, u1, u2, u3 = [
            lax.bitcast_convert_type(x_ref[...], jnp.uint32)
            for x_ref in (x0_ref, x1_ref, x2_ref, x3_ref)
        ]
        half = jnp.uint32(0x8000)
        mask = jnp.uint32(0xFFFF0000)
        p01 = ((u0 + half) >> 16) | ((u1 + half) & mask)
        p23 = ((u2 + half) >> 16) | ((u3 + half) & mask)
        p01 = jnp.transpose(lax.bitcast_convert_type(p01, jnp.float32))
        p23 = jnp.transpose(lax.bitcast_convert_type(p23, jnp.float32))
        out_ref[...] = jnp.concatenate([p01, p23], axis=1)

    return pl.pallas_call(
        body,
        grid=(NBLK,),
        in_specs=[
            pl.BlockSpec((D, BLKM), lambda i: (0, i)),
            pl.BlockSpec((D, BLKM), lambda i: (0, i + NBLK)),
            pl.BlockSpec((D, BLKM), lambda i: (0, i + 2 * NBLK)),
            pl.BlockSpec((D, BLKM),
                         lambda i: (0, jnp.minimum(i + 3 * NBLK, 122))),
            pl.BlockSpec((D, D), lambda i: (0, 0)),
        ],
        out_specs=pl.BlockSpec((BLKM, UD), lambda i: (i, 0)),
        out_shape=jax.ShapeDtypeStruct((QROWS, UD), jnp.float32),
    )(tableT, tableT, tableT, tableT, eye)


def _sc_gather(g, idx3):
    mesh = plsc.VectorSubcoreMesh(core_axis_name="c", subcore_axis_name="s")

    @functools.partial(
        pl.kernel,
        mesh=mesh,
        out_type=jax.ShapeDtypeStruct((B, UD), jnp.float32),
        scratch_types=[
            pltpu.VMEM((NCHUNK, CHUNK), jnp.int32),
            pltpu.VMEM((B_PER_W, UD), jnp.float32),
            pltpu.SemaphoreType.DMA,
        ],
    )
    def gather_kernel(g_hbm, idx_hbm, out_hbm, idx_v, rows_v, sem):
        wid = lax.axis_index("s") * NC + lax.axis_index("c")
        pltpu.sync_copy(idx_hbm.at[wid], idx_v)
        copies = [
            pltpu.async_copy(
                g_hbm.at[idx_v.at[j]],
                rows_v.at[pl.ds(j * CHUNK, CHUNK)],
                sem,
            )
            for j in range(NCHUNK)
        ]
        for c in copies:
            c.wait()
        pltpu.sync_copy(rows_v, out_hbm.at[pl.ds(wid * B_PER_W, B_PER_W)])

    return gather_kernel(g, idx3)


def _tc_mlp(user_emb, rows, quarter, Wu, bu2, W1u, W1i, b12, W2, b22, W3, b3s):
    grid = (B // BB,)

    def body(ue_ref, rows_ref, q_ref, Wu_ref, bu_ref, W1u_ref, W1i_ref,
             b1_ref, W2_ref, b2_ref, W3_ref, b3_ref, out_ref):
        dn = (((1,), (1,)), ((), ()))
        r = lax.bitcast_convert_type(rows_ref[...], jnp.uint32)
        q = q_ref[...]
        sel = jnp.where(q >= 2, r[:, D:], r[:, :D])
        lo = lax.bitcast_convert_type(
            (sel & 0xFFFF).astype(jnp.uint16), jnp.bfloat16)
        hi = lax.bitcast_convert_type(
            (sel >> 16).astype(jnp.uint16), jnp.bfloat16)
        i_emb = jnp.where((q & 1) == 1, hi, lo).astype(jnp.float32)
        u = lax.dot_general(ue_ref[...], Wu_ref[...], dn,
                            preferred_element_type=jnp.float32) + bu_ref[...]
        h1 = lax.dot_general(u, W1u_ref[...], dn,
                             preferred_element_type=jnp.float32)
        h1 = h1 + lax.dot_general(i_emb, W1i_ref[...], dn,
                                  preferred_element_type=jnp.float32)
        h1 = jnp.maximum(h1 + b1_ref[...], 0.0)
        h2 = lax.dot_general(h1, W2_ref[...], dn,
                             preferred_element_type=jnp.float32)
        h2 = jnp.maximum(h2 + b2_ref[...], 0.0)
        o = lax.dot_general(h2, W3_ref[...], dn,
                            preferred_element_type=jnp.float32)
        out_ref[...] = o[:, 0] + b3_ref[0]

    full = lambda shape: pl.BlockSpec(shape, lambda i: (0, 0))
    return pl.pallas_call(
        body,
        grid=grid,
        in_specs=[
            pl.BlockSpec((BB, UD), lambda i: (i, 0)),
            pl.BlockSpec((BB, UD), lambda i: (i, 0)),
            pl.BlockSpec((BB, 1), lambda i: (i, 0)),
            full((D, UD)),
            full((1, D)),
            full((D, D)),
            full((D, D)),
            full((1, D)),
            full((32, D)),
            full((1, 32)),
            full((1, 32)),
            pl.BlockSpec((1,), lambda i: (0,)),
        ],
        out_specs=pl.BlockSpec((BB,), lambda i: (i,)),
        out_shape=jax.ShapeDtypeStruct((B,), jnp.float32),
    )(user_emb, rows, quarter, Wu, bu2, W1u, W1i, b12, W2, b22, W3, b3s)


def kernel(user_emb, item_ids, Wu, bu, table, W1, b1, W2, b2, W3, b3):
    ids = item_ids.astype(jnp.int32)
    quarter = (ids // QROWS).astype(jnp.int32)
    idx = ids - quarter * QROWS
    idx3 = idx.reshape(NW, NCHUNK, CHUNK)

    g = _tc_relayout(table.T, jnp.eye(D, dtype=jnp.float32))
    rows = _sc_gather(g, idx3)

    W1u = W1[:, :D]
    W1i = W1[:, D:]
    return _tc_mlp(user_emb, rows, quarter.reshape(B, 1), Wu,
                   bu.reshape(1, D), W1u, W1i, b1.reshape(1, D), W2,
                   b2.reshape(1, 32), W3, b3)


# speedup vs baseline: 3.5376x; 1.0010x over previous
import functools

import jax
import jax.numpy as jnp
from jax import lax
from jax.experimental import pallas as pl
from jax.experimental.pallas import tpu as pltpu
from jax.experimental.pallas import tpu_sc as plsc

B = 16384
D = 64
UD = 128
N_ITEMS = 1000000

NC = 2
NS = 16
NW = NC * NS
B_PER_W = B // NW
CHUNK = 128
NCHUNK = B_PER_W // CHUNK

BLKM = 8192
NBLK = 31
QROWS = NBLK * BLKM
BB = 4096


def _tc_relayout(tableT, eye):
    def body(x0_ref, x1_ref, x2_ref, x3_ref, eye_ref, out_ref):
        del eye_ref
        u0, u1, u2, u3 = [
            lax.bitcast_convert_type(x_ref[...], jnp.uint32)
            for x_ref in (x0_ref, x1_ref, x2_ref, x3_ref)
        ]
        half = jnp.uint32(0x8000)
        mask = jnp.uint32(0xFFFF0000)
        p01 = ((u0 + half) >> 16) | ((u1 + half) & mask)
        p23 = ((u2 + half) >> 16) | ((u3 + half) & mask)
        p01 = jnp.transpose(lax.bitcast_convert_type(p01, jnp.float32))
        p23 = jnp.transpose(lax.bitcast_convert_type(p23, jnp.float32))
        out_ref[...] = jnp.concatenate([p01, p23], axis=1)

    return pl.pallas_call(
        body,
        grid=(NBLK,),
        in_specs=[
            pl.BlockSpec((D, BLKM), lambda i: (0, i)),
            pl.BlockSpec((D, BLKM), lambda i: (0, i + NBLK)),
            pl.BlockSpec((D, BLKM), lambda i: (0, i + 2 * NBLK)),
            pl.BlockSpec((D, BLKM),
                         lambda i: (0, jnp.minimum(i + 3 * NBLK, 122))),
            pl.BlockSpec((D, D), lambda i: (0, 0)),
        ],
        out_specs=pl.BlockSpec((BLKM, UD), lambda i: (i, 0)),
        out_shape=jax.ShapeDtypeStruct((QROWS, UD), jnp.float32),
    )(tableT, tableT, tableT, tableT, eye)


def _sc_gather(g, idx3):
    mesh = plsc.VectorSubcoreMesh(core_axis_name="c", subcore_axis_name="s")

    @functools.partial(
        pl.kernel,
        mesh=mesh,
        out_type=jax.ShapeDtypeStruct((B, UD), jnp.float32),
        scratch_types=[
            pltpu.VMEM((NCHUNK, CHUNK), jnp.int32),
            pltpu.VMEM((B_PER_W, UD), jnp.float32),
            pltpu.SemaphoreType.DMA,
        ],
    )
    def gather_kernel(g_hbm, idx_hbm, out_hbm, idx_v, rows_v, sem):
        wid = lax.axis_index("s") * NC + lax.axis_index("c")
        pltpu.sync_copy(idx_hbm.at[wid], idx_v)
        copies = [
            pltpu.async_copy(
                g_hbm.at[idx_v.at[j]],
                rows_v.at[pl.ds(j * CHUNK, CHUNK)],
                sem,
            )
            for j in range(NCHUNK)
        ]
        for c in copies:
            c.wait()
        pltpu.sync_copy(rows_v, out_hbm.at[pl.ds(wid * B_PER_W, B_PER_W)])

    return gather_kernel(g, idx3)


def _tc_mlp(user_emb, rows, quarter, Wu, bu2, W1u, W1i, b12, W2, b22, W3, b3s):
    grid = (B // BB,)

    def body(ue_ref, rows_ref, q_ref, Wu_ref, bu_ref, W1u_ref, W1i_ref,
             b1_ref, W2_ref, b2_ref, W3_ref, b3_ref, out_ref):
        dn = (((1,), (0,)), ((), ()))
        r = lax.bitcast_convert_type(rows_ref[...], jnp.uint32)
        q = q_ref[...]
        sel = jnp.where(q >= 2, r[:, D:], r[:, :D])
        lo = lax.bitcast_convert_type(
            (sel & 0xFFFF).astype(jnp.uint16), jnp.bfloat16)
        hi = lax.bitcast_convert_type(
            (sel >> 16).astype(jnp.uint16), jnp.bfloat16)
        i_emb = jnp.where((q & 1) == 1, hi, lo)
        u = lax.dot_general(ue_ref[...].astype(jnp.bfloat16),
                            Wu_ref[...].astype(jnp.bfloat16), dn,
                            preferred_element_type=jnp.float32) + bu_ref[...]
        h1 = lax.dot_general(u.astype(jnp.bfloat16),
                             W1u_ref[...].astype(jnp.bfloat16), dn,
                             preferred_element_type=jnp.float32)
        h1 = h1 + lax.dot_general(i_emb, W1i_ref[...].astype(jnp.bfloat16),
                                  dn, preferred_element_type=jnp.float32)
        h1 = jnp.maximum(h1 + b1_ref[...], 0.0).astype(jnp.bfloat16)
        h2 = lax.dot_general(h1, W2_ref[...].astype(jnp.bfloat16), dn,
                             preferred_element_type=jnp.float32)
        h2 = jnp.maximum(h2 + b2_ref[...], 0.0).astype(jnp.bfloat16)
        o = lax.dot_general(h2, W3_ref[...].astype(jnp.bfloat16), dn,
                            preferred_element_type=jnp.float32)
        out_ref[...] = o[:, 0] + b3_ref[0]

    full = lambda shape: pl.BlockSpec(shape, lambda i: (0, 0))
    return pl.pallas_call(
        body,
        grid=grid,
        in_specs=[
            pl.BlockSpec((BB, UD), lambda i: (i, 0)),
            pl.BlockSpec((BB, UD), lambda i: (i, 0)),
            pl.BlockSpec((BB, 1), lambda i: (i, 0)),
            full((UD, D)),
            full((1, D)),
            full((D, D)),
            full((D, D)),
            full((1, D)),
            full((D, 32)),
            full((1, 32)),
            full((32, 1)),
            pl.BlockSpec((1,), lambda i: (0,)),
        ],
        out_specs=pl.BlockSpec((BB,), lambda i: (i,)),
        out_shape=jax.ShapeDtypeStruct((B,), jnp.float32),
    )(user_emb, rows, quarter, Wu, bu2, W1u, W1i, b12, W2, b22, W3, b3s)


def kernel(user_emb, item_ids, Wu, bu, table, W1, b1, W2, b2, W3, b3):
    ids = item_ids.astype(jnp.int32)
    quarter = (ids // QROWS).astype(jnp.int32)
    idx = ids - quarter * QROWS
    idx3 = idx.reshape(NW, NCHUNK, CHUNK)

    g = _tc_relayout(table.T, jnp.eye(D, dtype=jnp.float32))
    rows = _sc_gather(g, idx3)

    W1u = W1[:, :D]
    W1i = W1[:, D:]
    return _tc_mlp(user_emb, rows, quarter.reshape(B, 1), Wu.T,
                   bu.reshape(1, D), W1u.T, W1i.T, b1.reshape(1, D), W2.T,
                   b2.reshape(1, 32), W3.T, b3)


# speedup vs baseline: 3.5629x; 1.0071x over previous
import functools

import jax
import jax.numpy as jnp
from jax import lax
from jax.experimental import pallas as pl
from jax.experimental.pallas import tpu as pltpu
from jax.experimental.pallas import tpu_sc as plsc

B = 16384
D = 64
UD = 128
N_ITEMS = 1000000

NC = 2
NS = 16
NW = NC * NS
B_PER_W = B // NW
CHUNK = 128
NCHUNK = B_PER_W // CHUNK

BLKM = 8192
NBLK = 31
QROWS = NBLK * BLKM
BB = 4096


def _tc_relayout(tableT):
    def body(x0_ref, x1_ref, x2_ref, x3_ref, out_ref):
        u0, u1, u2, u3 = [
            lax.bitcast_convert_type(x_ref[...], jnp.uint32)
            for x_ref in (x0_ref, x1_ref, x2_ref, x3_ref)
        ]
        half = jnp.uint32(0x8000)
        mask = jnp.uint32(0xFFFF0000)
        p01 = ((u0 + half) >> 16) | ((u1 + half) & mask)
        p23 = ((u2 + half) >> 16) | ((u3 + half) & mask)
        p01 = jnp.transpose(lax.bitcast_convert_type(p01, jnp.float32))
        p23 = jnp.transpose(lax.bitcast_convert_type(p23, jnp.float32))
        out_ref[...] = jnp.concatenate([p01, p23], axis=1)

    return pl.pallas_call(
        body,
        grid=(NBLK,),
        in_specs=[
            pl.BlockSpec((D, BLKM), lambda i: (0, i)),
            pl.BlockSpec((D, BLKM), lambda i: (0, i + NBLK)),
            pl.BlockSpec((D, BLKM), lambda i: (0, i + 2 * NBLK)),
            pl.BlockSpec((D, BLKM),
                         lambda i: (0, jnp.minimum(i + 3 * NBLK, 122))),
        ],
        out_specs=pl.BlockSpec((BLKM, UD), lambda i: (i, 0)),
        out_shape=jax.ShapeDtypeStruct((QROWS, UD), jnp.float32),
    )(tableT, tableT, tableT, tableT)


def _sc_gather(g, idx3):
    mesh = plsc.VectorSubcoreMesh(core_axis_name="c", subcore_axis_name="s")

    @functools.partial(
        pl.kernel,
        mesh=mesh,
        out_type=jax.ShapeDtypeStruct((B, UD), jnp.float32),
        scratch_types=[
            pltpu.VMEM((NCHUNK, CHUNK), jnp.int32),
            pltpu.VMEM((B_PER_W, UD), jnp.float32),
            pltpu.SemaphoreType.DMA,
        ],
    )
    def gather_kernel(g_hbm, idx_hbm, out_hbm, idx_v, rows_v, sem):
        wid = lax.axis_index("s") * NC + lax.axis_index("c")
        pltpu.sync_copy(idx_hbm.at[wid], idx_v)
        copies = [
            pltpu.async_copy(
                g_hbm.at[idx_v.at[j]],
                rows_v.at[pl.ds(j * CHUNK, CHUNK)],
                sem,
            )
            for j in range(NCHUNK)
        ]
        for c in copies:
            c.wait()
        pltpu.sync_copy(rows_v, out_hbm.at[pl.ds(wid * B_PER_W, B_PER_W)])

    return gather_kernel(g, idx3)


def _tc_mlp(user_emb, rows, quarter, Wu, bu2, W1u, W1i, b12, W2, b22, W3, b3s):
    grid = (B // BB,)

    def body(ue_ref, rows_ref, q_ref, Wu_ref, bu_ref, W1u_ref, W1i_ref,
             b1_ref, W2_ref, b2_ref, W3_ref, b3_ref, out_ref):
        dn = (((1,), (0,)), ((), ()))
        r = lax.bitcast_convert_type(rows_ref[...], jnp.uint32)
        q = q_ref[...]
        sel = jnp.where(q >= 2, r[:, D:], r[:, :D])
        lo = lax.bitcast_convert_type(
            (sel & 0xFFFF).astype(jnp.uint16), jnp.bfloat16)
        hi = lax.bitcast_convert_type(
            (sel >> 16).astype(jnp.uint16), jnp.bfloat16)
        i_emb = jnp.where((q & 1) == 1, hi, lo)
        u = lax.dot_general(ue_ref[...].astype(jnp.bfloat16),
                            Wu_ref[...].astype(jnp.bfloat16), dn,
                            preferred_element_type=jnp.float32) + bu_ref[...]
        h1 = lax.dot_general(u.astype(jnp.bfloat16),
                             W1u_ref[...].astype(jnp.bfloat16), dn,
                             preferred_element_type=jnp.float32)
        h1 = h1 + lax.dot_general(i_emb, W1i_ref[...].astype(jnp.bfloat16),
                                  dn, preferred_element_type=jnp.float32)
        h1 = jnp.maximum(h1 + b1_ref[...], 0.0).astype(jnp.bfloat16)
        h2 = lax.dot_general(h1, W2_ref[...].astype(jnp.bfloat16), dn,
                             preferred_element_type=jnp.float32)
        h2 = jnp.maximum(h2 + b2_ref[...], 0.0).astype(jnp.bfloat16)
        o = lax.dot_general(h2, W3_ref[...].astype(jnp.bfloat16), dn,
                            preferred_element_type=jnp.float32)
        out_ref[...] = o[:, 0] + b3_ref[0]

    full = lambda shape: pl.BlockSpec(shape, lambda i: (0, 0))
    return pl.pallas_call(
        body,
        grid=grid,
        in_specs=[
            pl.BlockSpec((BB, UD), lambda i: (i, 0)),
            pl.BlockSpec((BB, UD), lambda i: (i, 0)),
            pl.BlockSpec((BB, 1), lambda i: (i, 0)),
            full((UD, D)),
            full((1, D)),
            full((D, D)),
            full((D, D)),
            full((1, D)),
            full((D, 32)),
            full((1, 32)),
            full((32, 1)),
            pl.BlockSpec((1,), lambda i: (0,)),
        ],
        out_specs=pl.BlockSpec((BB,), lambda i: (i,)),
        out_shape=jax.ShapeDtypeStruct((B,), jnp.float32),
    )(user_emb, rows, quarter, Wu, bu2, W1u, W1i, b12, W2, b22, W3, b3s)


def kernel(user_emb, item_ids, Wu, bu, table, W1, b1, W2, b2, W3, b3):
    ids = item_ids.astype(jnp.int32)
    quarter = (ids // QROWS).astype(jnp.int32)
    idx = ids - quarter * QROWS
    idx3 = idx.reshape(NW, NCHUNK, CHUNK)

    g = _tc_relayout(table.T)
    rows = _sc_gather(g, idx3)

    W1u = W1[:, :D]
    W1i = W1[:, D:]
    return _tc_mlp(user_emb, rows, quarter.reshape(B, 1), Wu.T,
                   bu.reshape(1, D), W1u.T, W1i.T, b1.reshape(1, D), W2.T,
                   b2.reshape(1, 32), W3.T, b3)
